# baseline (speedup 1.0000x reference)
import jax
import jax.numpy as jnp
from jax.experimental import pallas as pl
from jax.experimental.pallas import tpu as pltpu

_B, _N, _M, _C1, _C2 = 8, 2048, 512, 256, 512
_H1, _H2 = 512, 256
_NB = 1024


def _fp_body(tT_ref, src_ref, tf_ref, sf_ref, W1a_ref, W1b_ref, W2_ref,
             b1_ref, b2_ref, out_ref):
    M = _M
    d2 = None
    for c in range(3):
        sc = src_ref[0, :, c:c + 1]
        tc = tT_ref[0, c:c + 1, :]
        diff = sc - tc
        d2 = diff * diff if c == 0 else d2 + diff * diff

    rs, scat_u = [], None
    for k in range(3):
        mv = jnp.min(d2, axis=0, keepdims=True)
        m_eq = d2 == mv
        r = 1.0 / (jnp.sqrt(mv) + 1e-8)
        rs.append(r)
        contrib = jnp.where(m_eq, r, 0.0)
        scat_u = contrib if k == 0 else scat_u + contrib
        if k < 2:
            d2 = jnp.where(m_eq, jnp.float32(jnp.inf), d2)

    inv_norm = 1.0 / (rs[0] + rs[1] + rs[2])
    scat = (scat_u * inv_norm).astype(jnp.bfloat16)

    S = jnp.dot(W1a_ref[...], sf_ref[0].astype(jnp.bfloat16),
                preferred_element_type=jnp.float32).astype(jnp.bfloat16)
    out1a = jnp.dot(S, scat, preferred_element_type=jnp.float32)
    out1b = jnp.dot(W1b_ref[...], tf_ref[0].astype(jnp.bfloat16),
                    preferred_element_type=jnp.float32)
    h1 = jnp.maximum(out1a + out1b + b1_ref[...], 0.0)
    out2 = jnp.dot(W2_ref[...], h1.astype(jnp.bfloat16),
                   preferred_element_type=jnp.float32)
    out_ref[0] = jnp.maximum(out2 + b2_ref[...], 0.0)


def kernel(target, source, target_feats, source_feats, W1, g1, b1, W2, g2, b2):
    B, N, M, C1, C2 = _B, _N, _M, _C1, _C2
    tT = jnp.transpose(target, (0, 2, 1))
    inv = 1.0 / jnp.sqrt(jnp.float32(1.0 + 1e-5))
    s1 = (g1 * inv).reshape(_H1, 1)
    s2 = (g2 * inv).reshape(_H2, 1)
    W1a = (W1[:, :C2] * s1).astype(jnp.bfloat16)
    W1b = (W1[:, C2:] * s1).astype(jnp.bfloat16)
    W2s = (W2 * s2).astype(jnp.bfloat16)
    b1c = b1.reshape(_H1, 1)
    b2c = b2.reshape(_H2, 1)

    nb = N // _NB
    grid = (B, nb)
    full = lambda shape: pl.BlockSpec(shape, lambda b, n: (0,) * len(shape))
    out = pl.pallas_call(
        _fp_body,
        grid=grid,
        in_specs=[
            pl.BlockSpec((1, 3, _NB), lambda b, n: (b, 0, n)),
            pl.BlockSpec((1, M, 3), lambda b, n: (b, 0, 0)),
            pl.BlockSpec((1, C1, _NB), lambda b, n: (b, 0, n)),
            pl.BlockSpec((1, C2, M), lambda b, n: (b, 0, 0)),
            full((_H1, C2)),
            full((_H1, C1)),
            full((_H2, _H1)),
            full((_H1, 1)),
            full((_H2, 1)),
        ],
        out_specs=pl.BlockSpec((1, _H2, _NB), lambda b, n: (b, 0, n)),
        out_shape=jax.ShapeDtypeStruct((B, _H2, N), jnp.float32),
    )(tT, source, target_feats, source_feats, W1a, W1b, W2s, b1c, b2c)
    return out


# speedup vs baseline: 1.1447x; 1.1447x over previous
import jax
import jax.numpy as jnp
from jax.experimental import pallas as pl
from jax.experimental.pallas import tpu as pltpu

_B, _N, _M, _C1, _C2 = 8, 2048, 512, 256, 512
_H1, _H2 = 512, 256
_NB = 1024


def _fp_body(tT_ref, src_ref, tf_ref, sf_ref, W1a_ref, W1b_ref, W2_ref,
             b1_ref, b2_ref, out_ref):
    M = _M
    src3 = src_ref[0]
    tT3 = tT_ref[0]
    st = jnp.dot(src3, tT3, preferred_element_type=jnp.float32)
    sn = jnp.sum(src3 * src3, axis=1, keepdims=True)
    tn = jnp.sum(tT3 * tT3, axis=0, keepdims=True)
    d2 = (sn + tn) - (st + st)

    rs, scat_u = [], None
    for k in range(3):
        mv = jnp.min(d2, axis=0, keepdims=True)
        m_eq = d2 == mv
        r = 1.0 / (jnp.sqrt(jnp.maximum(mv, 0.0)) + 1e-8)
        rs.append(r)
        contrib = jnp.where(m_eq, r, 0.0)
        scat_u = contrib if k == 0 else scat_u + contrib
        if k < 2:
            d2 = jnp.where(m_eq, jnp.float32(jnp.inf), d2)

    inv_norm = 1.0 / (rs[0] + rs[1] + rs[2])
    scat = scat_u * inv_norm

    S = jnp.dot(W1a_ref[...], sf_ref[0], preferred_element_type=jnp.float32)
    out1a = jnp.dot(S, scat, preferred_element_type=jnp.float32)
    out1b = jnp.dot(W1b_ref[...], tf_ref[0],
                    preferred_element_type=jnp.float32)
    h1 = jnp.maximum(out1a + out1b + b1_ref[...], 0.0)
    out2 = jnp.dot(W2_ref[...], h1, preferred_element_type=jnp.float32)
    out_ref[0] = jnp.maximum(out2 + b2_ref[...], 0.0)


def kernel(target, source, target_feats, source_feats, W1, g1, b1, W2, g2, b2):
    B, N, M, C1, C2 = _B, _N, _M, _C1, _C2
    tT = jnp.transpose(target, (0, 2, 1))
    inv = 1.0 / jnp.sqrt(jnp.float32(1.0 + 1e-5))
    s1 = (g1 * inv).reshape(_H1, 1)
    s2 = (g2 * inv).reshape(_H2, 1)
    W1a = W1[:, :C2] * s1
    W1b = W1[:, C2:] * s1
    W2s = W2 * s2
    b1c = b1.reshape(_H1, 1)
    b2c = b2.reshape(_H2, 1)

    nb = N // _NB
    grid = (B, nb)
    full = lambda shape: pl.BlockSpec(shape, lambda b, n: (0,) * len(shape))
    out = pl.pallas_call(
        _fp_body,
        grid=grid,
        in_specs=[
            pl.BlockSpec((1, 3, _NB), lambda b, n: (b, 0, n)),
            pl.BlockSpec((1, M, 3), lambda b, n: (b, 0, 0)),
            pl.BlockSpec((1, C1, _NB), lambda b, n: (b, 0, n)),
            pl.BlockSpec((1, C2, M), lambda b, n: (b, 0, 0)),
            full((_H1, C2)),
            full((_H1, C1)),
            full((_H2, _H1)),
            full((_H1, 1)),
            full((_H2, 1)),
        ],
        out_specs=pl.BlockSpec((1, _H2, _NB), lambda b, n: (b, 0, n)),
        out_shape=jax.ShapeDtypeStruct((B, _H2, N), jnp.float32),
    )(tT, source, target_feats, source_feats, W1a, W1b, W2s, b1c, b2c)
    return out
